# baseline (speedup 1.0000x reference)
import functools

import jax
import jax.numpy as jnp
from jax import lax
from jax.experimental import pallas as pl
from jax.experimental.pallas import tpu as pltpu
from jax.experimental.pallas import tpu_sc as plsc


def _sc_broadcast(pe_slice, batch_n):
    S, D = pe_slice.shape
    NC, NS = 2, 16
    NW = NC * NS
    rows_per_w = S // NW

    mesh = plsc.VectorSubcoreMesh(core_axis_name="c", subcore_axis_name="s")

    @functools.partial(
        pl.kernel,
        out_type=jax.ShapeDtypeStruct((S, batch_n, D), jnp.float32),
        mesh=mesh,
        scratch_types=[
            pltpu.VMEM((rows_per_w, D), jnp.float32),
            pltpu.SemaphoreType.DMA,
            pltpu.SemaphoreType.DMA,
        ],
    )
    def k(pe_hbm, out_hbm, rows_v, gsem, ssem):
        wid = lax.axis_index("s") * NC + lax.axis_index("c")
        base = wid * rows_per_w
        half = rows_per_w // 2
        gathers = [
            pltpu.make_async_copy(
                pe_hbm.at[pl.ds(base + j * half, half)],
                rows_v.at[pl.ds(j * half, half)],
                gsem,
            )
            for j in range(2)
        ]
        scatters = [
            pltpu.make_async_copy(
                rows_v.at[pl.ds(j * half, half)],
                out_hbm.at[pl.ds(base + j * half, half), n],
                ssem,
            )
            for j in range(2)
            for n in range(batch_n)
        ]
        gathers[0].start()
        gathers[1].start()
        gathers[0].wait()
        for c in scatters[:batch_n]:
            c.start()
        gathers[1].wait()
        for c in scatters[batch_n:]:
            c.start()
        for c in scatters:
            c.wait()

    return k(pe_slice)


def _tc_broadcast(pe_slice, batch_n):
    S, D = pe_slice.shape
    blk = 256

    def body(pe_ref, out_ref):
        x = pe_ref[...]
        out_ref[...] = jnp.broadcast_to(x[:, None, :], (blk, batch_n, D))

    return pl.pallas_call(
        body,
        grid=(S // blk,),
        in_specs=[pl.BlockSpec((blk, D), lambda i: (i, 0))],
        out_specs=pl.BlockSpec((blk, batch_n, D), lambda i: (i, 0, 0)),
        out_shape=jax.ShapeDtypeStruct((S, batch_n, D), jnp.float32),
    )(pe_slice)


def kernel(z, pos_embed):
    batch_n = z.shape[1]
    S = pos_embed.shape[0]
    s_sc = S // 2
    out_sc = _sc_broadcast(pos_embed[:s_sc], batch_n)
    out_tc = _tc_broadcast(pos_embed[s_sc:], batch_n)
    return jnp.concatenate([out_sc, out_tc], axis=0)


# speedup vs baseline: 2.3838x; 2.3838x over previous
import functools

import jax
import jax.numpy as jnp
from jax import lax
from jax.experimental import pallas as pl
from jax.experimental.pallas import tpu as pltpu
from jax.experimental.pallas import tpu_sc as plsc


def _const_embed_sc(pos_embed, batch_n):
    S, D = pos_embed.shape
    NC, NS = 2, 16
    NW = NC * NS
    rows_per_w = S // NW

    mesh = plsc.VectorSubcoreMesh(core_axis_name="c", subcore_axis_name="s")

    @functools.partial(
        pl.kernel,
        out_type=jax.ShapeDtypeStruct((S, batch_n, D), jnp.float32),
        mesh=mesh,
        scratch_types=[
            pltpu.VMEM((rows_per_w, D), jnp.float32),
            pltpu.SemaphoreType.DMA,
            pltpu.SemaphoreType.DMA,
        ],
    )
    def k(pe_hbm, out_hbm, rows_v, gsem, ssem):
        wid = lax.axis_index("c") * NS + lax.axis_index("s")
        base = wid * rows_per_w
        half = rows_per_w // 2
        gathers = [
            pltpu.make_async_copy(
                pe_hbm.at[pl.ds(base + j * half, half)],
                rows_v.at[pl.ds(j * half, half)],
                gsem,
            )
            for j in range(2)
        ]
        scatters = [
            pltpu.make_async_copy(
                rows_v.at[pl.ds(j * half, half)],
                out_hbm.at[pl.ds(base + j * half, half), n],
                ssem,
            )
            for j in range(2)
            for n in range(batch_n)
        ]
        gathers[0].start()
        gathers[1].start()
        gathers[0].wait()
        for c in scatters[:batch_n]:
            c.start()
        gathers[1].wait()
        for c in scatters[batch_n:]:
            c.start()
        for c in scatters:
            c.wait()

    return k(pos_embed)


def kernel(z, pos_embed):
    return _const_embed_sc(pos_embed, z.shape[1])
